# baseline (speedup 1.0000x reference)
import functools

import jax
import jax.numpy as jnp
import numpy as np
from jax import lax
from jax.experimental import pallas as pl
from jax.experimental.pallas import tpu as pltpu
from jax.experimental.pallas import tpu_sc as plsc

B, C, T = 64, 8, 32768
K, DT = 16, 1024
W = B * C * K
L = 16
NTILES = 32
WPT = W // NTILES
G = 16
NG = WPT // G
NCH = 9
SWIN = NCH * 128
XROWS = B * C * T // 1024

_MASK32 = np.uint64(0xFFFFFFFF)


def _threefry2x32(k1, k2, x0, x1):
    rots = (np.uint64([13, 15, 26, 6]), np.uint64([17, 29, 16, 24]))
    ks = [k1, k2, k1 ^ k2 ^ np.uint64(0x1BD11BDA)]
    x = [(x0 + ks[0]) & _MASK32, (x1 + ks[1]) & _MASK32]
    kcur = [ks[1], ks[2], ks[0]]
    rcur = [rots[0], rots[1]]
    for i in range(5):
        for r in rcur[0]:
            x[0] = (x[0] + x[1]) & _MASK32
            x[1] = ((x[1] << r) | (x[1] >> (np.uint64(32) - r))) & _MASK32
            x[1] = x[0] ^ x[1]
        x = [(x[0] + kcur[0]) & _MASK32,
             (x[1] + kcur[1] + np.uint64(i + 1)) & _MASK32]
        kcur = kcur[1:] + kcur[:1]
        rcur = rcur[1:] + rcur[:1]
    return x[0], x[1]


def _draw_offsets():
    z = np.uint64(0)
    ok1, ok2 = _threefry2x32(z, z, np.uint64(0), np.uint64(1))
    b1, b2 = _threefry2x32(ok1, ok2, np.uint64([0, 0]), np.uint64([0, 1]))
    n = B * C * K
    chi = np.arange(n, dtype=np.uint64) >> np.uint64(32)
    clo = np.arange(n, dtype=np.uint64) & _MASK32
    r1, r2 = _threefry2x32(b1[0], b2[0], chi, clo)
    higher_bits = r1 ^ r2
    r1, r2 = _threefry2x32(b1[1], b2[1], chi, clo)
    lower_bits = r1 ^ r2
    span = np.uint64(T - DT + 1)
    mult = np.uint64(2 ** 16) % span
    mult = (mult * mult) % span
    off = ((higher_bits % span) * mult + (lower_bits % span)) % span
    return off.astype(np.int64).reshape(B, C, K)


def _build_tables():
    off = _draw_offsets()
    w = np.arange(W, dtype=np.int64)
    b = w >> 7
    c = (w >> 4) & 7
    k = w & 15
    o = off[b, c, k]
    s = (b * 256 + (o >> 7)) * 1024 + c * 128
    s_splat = np.broadcast_to(s[:, None], (W, L))
    rvec = (o & 127)[:, None] + np.arange(L, dtype=np.int64)
    return s_splat.astype(np.int32).copy(), rvec.astype(np.int32)


_S, _RV = _build_tables()


def _sample_body(x, s_hbm, rv_hbm, out, s_v, rv_v, *rest):
    stage = [list(rest[0:G]), list(rest[G:2 * G])]
    outb = list(rest[2 * G:2 * G + 2])
    gsems = list(rest[2 * G + 2:2 * G + 4])
    osems = list(rest[2 * G + 4:2 * G + 6])
    wid = lax.axis_index("s") * 2 + lax.axis_index("c")
    w0 = wid * WPT
    pltpu.sync_copy(s_hbm.at[pl.ds(w0, WPT)], s_v)
    pltpu.sync_copy(rv_hbm.at[pl.ds(w0, WPT)], rv_v)

    def issue(t, slot):
        for g in range(G):
            s = lax.reduce_max(s_v[t * G + g], (0,))
            row = s >> 10
            col = pl.multiple_of(s & 1023, 128)
            pltpu.async_copy(x.at[pl.ds(row, NCH), pl.ds(col, 128)],
                             stage[slot][g], gsems[slot])

    def drain(slot):
        for g in range(G):
            pltpu.make_async_copy(x.at[pl.ds(0, NCH), pl.ds(0, 128)],
                                  stage[slot][g], gsems[slot]).wait()

    def flush_dst(t):
        return out.at[pl.ds((w0 + t * G) * DT, G * DT)]

    def process(i, t, slot):
        @pl.when(i > 0)
        def _():
            pltpu.make_async_copy(outb[slot], flush_dst(t), osems[slot]).wait()

        for g in range(G):
            pltpu.make_async_copy(x.at[pl.ds(0, NCH), pl.ds(0, 128)],
                                  stage[slot][g], gsems[slot]).wait()
            rvec = rv_v[t * G + g]
            zero = jnp.zeros((L,), jnp.int32)
            for j in range(DT // L):
                v = plsc.load_gather(stage[slot][g], [zero, rvec + (j * L)])
                a, mm = j >> 3, j & 7
                outb[slot][pl.ds((g >> 3) * 8192 + a * DT + (g & 7) * 128 + mm * L, L)] = v
        pltpu.async_copy(outb[slot], flush_dst(t), osems[slot])

        @pl.when(t + 2 < NG)
        def _():
            issue(t + 2, slot)

    issue(0, 0)
    issue(1, 1)

    def body(i, carry):
        process(i, 2 * i, 0)
        process(i, 2 * i + 1, 1)
        return carry

    lax.fori_loop(0, NG // 2, body, 0)
    pltpu.make_async_copy(outb[0], flush_dst(0), osems[0]).wait()
    pltpu.make_async_copy(outb[1], flush_dst(0), osems[1]).wait()


@functools.cache
def _build_kernel():
    return functools.partial(
        pl.kernel,
        mesh=plsc.VectorSubcoreMesh(core_axis_name="c", subcore_axis_name="s"),
        out_type=jax.ShapeDtypeStruct((W * DT,), jnp.float32),
        compiler_params=pltpu.CompilerParams(needs_layout_passes=False,
                                             use_tc_tiling_on_sc=False,
                                             disable_bounds_checks=True),
        scratch_types=(
            [pltpu.VMEM((WPT, L), jnp.int32)] * 2
            + [pltpu.VMEM((NCH, 128), jnp.float32)] * (2 * G)
            + [pltpu.VMEM((G * DT,), jnp.float32)] * 2
            + [pltpu.SemaphoreType.DMA] * 4
        ),
    )(_sample_body)


def kernel(input):
    xt = input.reshape(B, C, 256, 128).transpose(0, 2, 1, 3)
    out = _build_kernel()(xt.reshape(XROWS, 1024), jnp.asarray(_S),
                          jnp.asarray(_RV))
    return (out.reshape(B, 16, 8, 8, 128).transpose(0, 1, 3, 2, 4)
            .reshape(B, C * K, DT))


# speedup vs baseline: 1.1284x; 1.1284x over previous
import functools

import jax
import jax.numpy as jnp
import numpy as np
from jax import lax
from jax.experimental import pallas as pl
from jax.experimental.pallas import tpu as pltpu
from jax.experimental.pallas import tpu_sc as plsc

B, C, T = 64, 8, 32768
K, DT = 16, 1024
W = B * C * K
L = 16
NTILES = 32
WPT = W // NTILES
G = 8
NG = WPT // G
NCH = 9
SWIN = NCH * 128
XROWS = B * C * T // 1024

_MASK32 = np.uint64(0xFFFFFFFF)


def _threefry2x32(k1, k2, x0, x1):
    rots = (np.uint64([13, 15, 26, 6]), np.uint64([17, 29, 16, 24]))
    ks = [k1, k2, k1 ^ k2 ^ np.uint64(0x1BD11BDA)]
    x = [(x0 + ks[0]) & _MASK32, (x1 + ks[1]) & _MASK32]
    kcur = [ks[1], ks[2], ks[0]]
    rcur = [rots[0], rots[1]]
    for i in range(5):
        for r in rcur[0]:
            x[0] = (x[0] + x[1]) & _MASK32
            x[1] = ((x[1] << r) | (x[1] >> (np.uint64(32) - r))) & _MASK32
            x[1] = x[0] ^ x[1]
        x = [(x[0] + kcur[0]) & _MASK32,
             (x[1] + kcur[1] + np.uint64(i + 1)) & _MASK32]
        kcur = kcur[1:] + kcur[:1]
        rcur = rcur[1:] + rcur[:1]
    return x[0], x[1]


def _draw_offsets():
    z = np.uint64(0)
    ok1, ok2 = _threefry2x32(z, z, np.uint64(0), np.uint64(1))
    b1, b2 = _threefry2x32(ok1, ok2, np.uint64([0, 0]), np.uint64([0, 1]))
    n = B * C * K
    chi = np.arange(n, dtype=np.uint64) >> np.uint64(32)
    clo = np.arange(n, dtype=np.uint64) & _MASK32
    r1, r2 = _threefry2x32(b1[0], b2[0], chi, clo)
    higher_bits = r1 ^ r2
    r1, r2 = _threefry2x32(b1[1], b2[1], chi, clo)
    lower_bits = r1 ^ r2
    span = np.uint64(T - DT + 1)
    mult = np.uint64(2 ** 16) % span
    mult = (mult * mult) % span
    off = ((higher_bits % span) * mult + (lower_bits % span)) % span
    return off.astype(np.int64).reshape(B, C, K)


def _build_tables():
    off = _draw_offsets()
    w = np.arange(W, dtype=np.int64)
    b = w >> 7
    c = (w >> 4) & 7
    k = w & 15
    o = off[b, c, k]
    s = (b * 256 + (o >> 7)) * 1024 + c * 128
    s_splat = np.broadcast_to(s[:, None], (W, L))
    rvec = (o & 127)[:, None] + np.arange(L, dtype=np.int64)
    return s_splat.astype(np.int32).copy(), rvec.astype(np.int32)


_S, _RV = _build_tables()


def _sample_body(x, s_hbm, rv_hbm, out, s_v, rv_v, *rest):
    stage = [list(rest[0:G]), list(rest[G:2 * G])]
    outb = list(rest[2 * G:2 * G + 2])
    gsems = list(rest[2 * G + 2:2 * G + 4])
    osems = list(rest[2 * G + 4:2 * G + 6])
    wid = lax.axis_index("s") * 2 + lax.axis_index("c")
    w0 = wid * WPT
    pltpu.sync_copy(s_hbm.at[pl.ds(w0, WPT)], s_v)
    pltpu.sync_copy(rv_hbm.at[pl.ds(w0, WPT)], rv_v)

    def issue(t, slot):
        for g in range(G):
            s = lax.reduce_max(s_v[t * G + g], (0,))
            row = s >> 10
            col = pl.multiple_of(s & 1023, 128)
            pltpu.async_copy(x.at[pl.ds(row, NCH), pl.ds(col, 128)],
                             stage[slot][g], gsems[slot])

    def drain(slot):
        for g in range(G):
            pltpu.make_async_copy(x.at[pl.ds(0, NCH), pl.ds(0, 128)],
                                  stage[slot][g], gsems[slot]).wait()

    def flush_dst(t):
        return out.at[pl.ds((w0 + t * G) * DT, G * DT)]

    def process(i, t, slot):
        @pl.when(i > 0)
        def _():
            pltpu.make_async_copy(outb[slot], flush_dst(t), osems[slot]).wait()

        drain(slot)
        for g in range(G):
            rvec = rv_v[t * G + g]
            zero = jnp.zeros((L,), jnp.int32)
            for j in range(DT // L):
                v = plsc.load_gather(stage[slot][g], [zero, rvec + (j * L)])
                a, mm = j >> 3, j & 7
                outb[slot][pl.ds(a * DT + g * 128 + mm * L, L)] = v
        pltpu.async_copy(outb[slot], flush_dst(t), osems[slot])

        @pl.when(t + 2 < NG)
        def _():
            issue(t + 2, slot)

    issue(0, 0)
    issue(1, 1)

    def body(i, carry):
        process(i, 2 * i, 0)
        process(i, 2 * i + 1, 1)
        return carry

    lax.fori_loop(0, NG // 2, body, 0)
    pltpu.make_async_copy(outb[0], flush_dst(0), osems[0]).wait()
    pltpu.make_async_copy(outb[1], flush_dst(0), osems[1]).wait()


@functools.cache
def _build_kernel():
    return functools.partial(
        pl.kernel,
        mesh=plsc.VectorSubcoreMesh(core_axis_name="c", subcore_axis_name="s"),
        out_type=jax.ShapeDtypeStruct((W * DT,), jnp.float32),
        compiler_params=pltpu.CompilerParams(needs_layout_passes=False,
                                             use_tc_tiling_on_sc=False,
                                             disable_bounds_checks=True),
        scratch_types=(
            [pltpu.VMEM((WPT, L), jnp.int32)] * 2
            + [pltpu.VMEM((NCH, 128), jnp.float32)] * (2 * G)
            + [pltpu.VMEM((G * DT,), jnp.float32)] * 2
            + [pltpu.SemaphoreType.DMA] * 4
        ),
    )(_sample_body)


def kernel(input):
    xt = input.reshape(B, C, 256, 128).transpose(0, 2, 1, 3)
    out = _build_kernel()(xt.reshape(XROWS, 1024), jnp.asarray(_S),
                          jnp.asarray(_RV))
    return (out.reshape(B, 16, 8, 8, 128).transpose(0, 1, 3, 2, 4)
            .reshape(B, C * K, DT))
